# baseline (speedup 1.0000x reference)
import functools

import jax
import jax.numpy as jnp
from jax.experimental import pallas as pl
from jax.experimental.pallas import tpu as pltpu


def _router_kernel(x_ref, wr_ref, br_ref, gates_ref):
    x = x_ref[...]
    wr = wr_ref[...]
    logits = (
        jnp.dot(x, wr, preferred_element_type=jnp.float32)
        + br_ref[...][None, :]
    )
    m = jnp.max(logits, axis=-1, keepdims=True)
    ex = jnp.exp(logits - m)
    p = ex / jnp.sum(ex, axis=-1, keepdims=True)

    S, E = p.shape
    lane = jax.lax.broadcasted_iota(jnp.int32, (S, E), 1)
    i1 = jnp.argmax(p, axis=-1)[:, None]
    m1 = jnp.max(p, axis=-1, keepdims=True)
    sel1 = lane == i1
    p_rest = jnp.where(sel1, -1.0, p)
    i2 = jnp.argmax(p_rest, axis=-1)[:, None]
    m2 = jnp.max(p_rest, axis=-1, keepdims=True)
    denom = m1 + m2
    gates = jnp.where(sel1, m1 / denom,
                      jnp.where(lane == i2, m2 / denom, 0.0))
    gates_ref[...] = gates


def _expert_kernel(gates_ref, x_ref, w1_ref, b1_ref, w2_ref, b2_ref, out_ref):
    e = pl.program_id(0)
    s = pl.program_id(1)
    bs = x_ref.shape[0]

    @pl.when((e == 0) & (s == 0))
    def _init():
        out_ref[...] = jnp.zeros_like(out_ref)

    x = x_ref[...]
    h = jnp.dot(x, w1_ref[0], preferred_element_type=jnp.float32)
    h = jnp.maximum(h + b1_ref[0], 0.0).astype(jnp.bfloat16)
    o = jnp.dot(h, w2_ref[0], preferred_element_type=jnp.float32)
    o = o + b2_ref[0]

    lane = jax.lax.broadcasted_iota(jnp.int32, gates_ref.shape, 1)
    g = jnp.sum(jnp.where(lane == e, gates_ref[...], 0.0),
                axis=1, keepdims=True)
    out_ref[pl.ds(s * bs, bs), :] += o * g


@functools.partial(jax.jit, static_argnames=())
def kernel(x, Wr, br, W1, b1, W2, b2):
    B, S, D = x.shape
    E = Wr.shape[1]
    H = W1.shape[2]
    xs = x.reshape(B * S, D)

    gates = pl.pallas_call(
        _router_kernel,
        out_shape=jax.ShapeDtypeStruct((B * S, E), jnp.float32),
    )(xs, Wr, br)

    BS = 512
    n_s = (B * S) // BS
    x16 = xs.astype(jnp.bfloat16)
    w1_16 = W1.astype(jnp.bfloat16)
    w2_16 = W2.astype(jnp.bfloat16)

    out = pl.pallas_call(
        _expert_kernel,
        grid=(E, n_s),
        in_specs=[
            pl.BlockSpec((BS, E), lambda e, s: (s, 0)),
            pl.BlockSpec((BS, D), lambda e, s: (s, 0)),
            pl.BlockSpec((1, D, H), lambda e, s: (e, 0, 0)),
            pl.BlockSpec((1, 1, H), lambda e, s: (e, 0, 0)),
            pl.BlockSpec((1, H, D), lambda e, s: (e, 0, 0)),
            pl.BlockSpec((1, 1, D), lambda e, s: (e, 0, 0)),
        ],
        out_specs=pl.BlockSpec((B * S, D), lambda e, s: (0, 0)),
        out_shape=jax.ShapeDtypeStruct((B * S, D), jnp.float32),
        compiler_params=pltpu.CompilerParams(
            dimension_semantics=("arbitrary", "arbitrary"),
        ),
    )(gates, x16, w1_16, b1.reshape(E, 1, H), w2_16, b2.reshape(E, 1, D))

    return out.reshape(B, S, D)
